# initial kernel scaffold (unmeasured)
import functools

import jax
import jax.numpy as jnp
import numpy as np
from jax import lax
from jax.experimental import pallas as pl
from jax.experimental.pallas import tpu as pltpu

N_DEV = 4
SQ = 4096
SEQ_SH = 1024
D = 1024
H = 8
DH = 128
QB = 1024
SCALE = 0.08838834764831843


def _rope_tables():
    inv = 1.0 / (10000.0 ** (np.arange(0, DH, 2) / DH))
    pos = np.arange(SQ)[:, None] * inv[None, :]
    cos = np.repeat(np.cos(pos), 2, axis=-1).astype(np.float32)
    sin = np.repeat(np.sin(pos), 2, axis=-1).astype(np.float32)
    return cos, sin


_COS, _SIN = _rope_tables()


def _body(x_ref, wq_ref, wk_ref, wv_ref, wo_ref, cos_ref, sin_ref, out_ref,
          xf_ref, acc_ref, snd_ref, rcv_ref, ag_ss, ag_rs, rs_ss, rs_rs):
    my = lax.axis_index("i")
    right = lax.rem(my + 1, N_DEV)
    left = lax.rem(my + N_DEV - 1, N_DEV)

    barrier = pltpu.get_barrier_semaphore()
    for nbr in (left, right):
        pl.semaphore_signal(barrier, inc=1, device_id=(nbr,),
                            device_id_type=pl.DeviceIdType.MESH)
    pl.semaphore_wait(barrier, 2)

    xf_ref[pl.ds(my * SEQ_SH, SEQ_SH), :] = x_ref[...]
    for h in range(N_DEV - 1):
        src_o = lax.rem(my - h + N_DEV, N_DEV)
        rdma = pltpu.make_async_remote_copy(
            src_ref=xf_ref.at[pl.ds(src_o * SEQ_SH, SEQ_SH)],
            dst_ref=xf_ref.at[pl.ds(src_o * SEQ_SH, SEQ_SH)],
            send_sem=ag_ss.at[h],
            recv_sem=ag_rs.at[h],
            device_id=(right,),
            device_id_type=pl.DeviceIdType.MESH,
        )
        rdma.start()
        rdma.wait()

    cos = cos_ref[...]
    sin = sin_ref[...]
    lane = lax.broadcasted_iota(jnp.int32, (SQ, DH), 1)
    even = (lane % 2) == 0

    def rope(t):
        t_rot = jnp.where(even, -pltpu.roll(t, DH - 1, 1), pltpu.roll(t, 1, 1))
        return t * cos + t_rot * sin

    xf = xf_ref[...]
    for h in range(H):
        hs = slice(h * DH, (h + 1) * DH)
        q = rope(jnp.dot(xf, wq_ref[:, hs], preferred_element_type=jnp.float32))
        k = rope(jnp.dot(xf, wk_ref[:, hs], preferred_element_type=jnp.float32))
        v = jnp.dot(xf, wv_ref[:, hs], preferred_element_type=jnp.float32)
        wo_h = wo_ref[hs, :]
        for qb in range(SQ // QB):
            qsl = slice(qb * QB, (qb + 1) * QB)
            s = lax.dot_general(q[qsl], k, (((1,), (1,)), ((), ())),
                                preferred_element_type=jnp.float32) * SCALE
            s = s - jnp.max(s, axis=1, keepdims=True)
            e = jnp.exp(s)
            w = e / jnp.sum(e, axis=1, keepdims=True)
            ctx = jnp.dot(w, v, preferred_element_type=jnp.float32)
            contrib = jnp.dot(ctx, wo_h, preferred_element_type=jnp.float32)
            if h == 0:
                acc_ref[qsl, :] = contrib
            else:
                acc_ref[qsl, :] = acc_ref[qsl, :] + contrib

    for st in range(N_DEV - 1):
        c = lax.rem(my - 1 - st + N_DEV, N_DEV)
        chunk = acc_ref[pl.ds(c * SEQ_SH, SEQ_SH), :]
        if st == 0:
            snd_ref[...] = chunk
        else:
            snd_ref[...] = chunk + rcv_ref[st - 1]
        rdma = pltpu.make_async_remote_copy(
            src_ref=snd_ref,
            dst_ref=rcv_ref.at[st],
            send_sem=rs_ss.at[st],
            recv_sem=rs_rs.at[st],
            device_id=(right,),
            device_id_type=pl.DeviceIdType.MESH,
        )
        rdma.start()
        rdma.wait()
    out_ref[...] = acc_ref[pl.ds(my * SEQ_SH, SEQ_SH), :] + rcv_ref[N_DEV - 2]

    @functools.partial(pl.run_scoped, second_barrier=pltpu.SemaphoreType.REGULAR)
    def _(second_barrier):
        for nbr in (left, right):
            pl.semaphore_signal(second_barrier, inc=1, device_id=(nbr,),
                                device_id_type=pl.DeviceIdType.MESH)
        pl.semaphore_wait(second_barrier, 2)


def kernel(x, Wq, Wk, Wv, Wo):
    x2d = x.reshape(SEQ_SH, D)
    out = pl.pallas_call(
        _body,
        out_shape=jax.ShapeDtypeStruct((SEQ_SH, D), jnp.float32),
        in_specs=[pl.BlockSpec(memory_space=pltpu.VMEM)] * 7,
        out_specs=pl.BlockSpec(memory_space=pltpu.VMEM),
        scratch_shapes=[
            pltpu.VMEM((SQ, D), jnp.float32),
            pltpu.VMEM((SQ, D), jnp.float32),
            pltpu.VMEM((SEQ_SH, D), jnp.float32),
            pltpu.VMEM((N_DEV - 1, SEQ_SH, D), jnp.float32),
            pltpu.SemaphoreType.DMA((N_DEV - 1,)),
            pltpu.SemaphoreType.DMA((N_DEV - 1,)),
            pltpu.SemaphoreType.DMA((N_DEV - 1,)),
            pltpu.SemaphoreType.DMA((N_DEV - 1,)),
        ],
        compiler_params=pltpu.CompilerParams(collective_id=0),
    )(x2d, Wq, Wk, Wv, Wo, jnp.asarray(_COS), jnp.asarray(_SIN))
    return out.reshape(1, SEQ_SH, D)


# baseline (device time: 887344 ns/iter reference)
import functools

import jax
import jax.numpy as jnp
import numpy as np
from jax import lax
from jax.experimental import pallas as pl
from jax.experimental.pallas import tpu as pltpu

N_DEV = 4
SQ = 4096
CH = 1024
D = 1024
H = 8
DH = 128
QB = 512
SCALE = 0.08838834764831843


def _rope_tables():
    inv = 1.0 / (10000.0 ** (np.arange(0, DH, 2) / DH))
    pos = np.arange(SQ)[:, None] * inv[None, :]
    cos = np.repeat(np.cos(pos), 2, axis=-1).astype(np.float32)
    sin = np.repeat(np.sin(pos), 2, axis=-1).astype(np.float32)
    return cos, sin


_COS, _SIN = _rope_tables()


def _body(x_ref, wq_ref, wk_ref, wv_ref, wo_ref, cos_ref, sin_ref,
          out_ref, xf, ctx_hbm, acc, rcv,
          bufa, bufb, bufc, wbuf, qs, ks, vs, ctxh,
          ldma, ag_ss, ag_rs, rs_ss, rs_rs):
    my = lax.axis_index("i")
    right = lax.rem(my + 1, N_DEV)
    left = lax.rem(my + N_DEV - 1, N_DEV)

    barrier = pltpu.get_barrier_semaphore()
    for nbr in (left, right):
        pl.semaphore_signal(barrier, inc=1, device_id=(nbr,),
                            device_id_type=pl.DeviceIdType.MESH)
    pl.semaphore_wait(barrier, 2)

    cp = pltpu.make_async_copy(x_ref, xf.at[pl.ds(my * CH, CH)], ldma.at[0])
    cp.start()
    cp.wait()
    for hop in range(N_DEV - 1):
        src_o = lax.rem(my - hop + N_DEV, N_DEV)
        rdma = pltpu.make_async_remote_copy(
            src_ref=xf.at[pl.ds(src_o * CH, CH)],
            dst_ref=xf.at[pl.ds(src_o * CH, CH)],
            send_sem=ag_ss.at[hop],
            recv_sem=ag_rs.at[hop],
            device_id=(right,),
            device_id_type=pl.DeviceIdType.MESH,
        )
        rdma.start()
        rdma.wait()

    lane = lax.broadcasted_iota(jnp.int32, (CH, DH), 1)
    even = (lane % 2) == 0

    def rope(t, c, s):
        t_rot = jnp.where(even, -pltpu.roll(t, DH - 1, 1),
                          pltpu.roll(t, 1, 1))
        return t * c + t_rot * s

    def head_body(h, carry):
        hsl = pl.ds(h * DH, DH)
        cws = []
        for i, w_ref in enumerate((wq_ref, wk_ref, wv_ref)):
            cw = pltpu.make_async_copy(
                w_ref.at[:, hsl], wbuf.at[:, pl.ds(i * DH, DH)], ldma.at[i])
            cw.start()
            cws.append(cw)
        for cw in cws:
            cw.wait()

        def chunk_body(c, carry):
            cx = pltpu.make_async_copy(xf.at[pl.ds(c * CH, CH)], bufa,
                                       ldma.at[3])
            cx.start()
            cx.wait()
            xc = bufa[...]
            csl = pl.ds(c * CH, CH)
            qc = jnp.dot(xc, wbuf[:, 0:DH], preferred_element_type=jnp.float32)
            kc = jnp.dot(xc, wbuf[:, DH:2 * DH],
                         preferred_element_type=jnp.float32)
            vc = jnp.dot(xc, wbuf[:, 2 * DH:3 * DH],
                         preferred_element_type=jnp.float32)
            qs[csl, :] = rope(qc, cos_ref[csl, :], sin_ref[csl, :])
            ks[csl, :] = rope(kc, cos_ref[csl, :], sin_ref[csl, :])
            vs[csl, :] = vc
            return carry

        lax.fori_loop(0, N_DEV, chunk_body, 0)

        def qb_body(qb, carry):
            qsl = pl.ds(qb * QB, QB)
            s = lax.dot_general(qs[qsl, :], ks[...], (((1,), (1,)), ((), ())),
                                preferred_element_type=jnp.float32) * SCALE
            s = s - jnp.max(s, axis=1, keepdims=True)
            e = jnp.exp(s)
            w = e / jnp.sum(e, axis=1, keepdims=True)
            ctxh[qsl, :] = jnp.dot(w, vs[...],
                                   preferred_element_type=jnp.float32)
            return carry

        lax.fori_loop(0, SQ // QB, qb_body, 0)
        co = pltpu.make_async_copy(ctxh, ctx_hbm.at[:, hsl], ldma.at[3])
        co.start()
        co.wait()
        return carry

    lax.fori_loop(0, H, head_body, 0)

    cw = pltpu.make_async_copy(wo_ref, bufb, ldma.at[0])
    cw.start()
    cw.wait()
    for c in range(N_DEV):
        ci = pltpu.make_async_copy(ctx_hbm.at[pl.ds(c * CH, CH)], bufa,
                                   ldma.at[1])
        ci.start()
        ci.wait()
        bufc[...] = jnp.dot(bufa[...], bufb[...],
                            preferred_element_type=jnp.float32)
        co = pltpu.make_async_copy(bufc, acc.at[pl.ds(c * CH, CH)],
                                   ldma.at[2])
        co.start()
        co.wait()

    for st in range(N_DEV - 1):
        c = lax.rem(my - 1 - st + N_DEV, N_DEV)
        if st == 0:
            src = acc.at[pl.ds(c * CH, CH)]
        else:
            ca = pltpu.make_async_copy(rcv.at[st - 1], bufa, ldma.at[0])
            cb = pltpu.make_async_copy(acc.at[pl.ds(c * CH, CH)], bufb,
                                       ldma.at[1])
            ca.start()
            cb.start()
            ca.wait()
            cb.wait()
            bufc[...] = bufa[...] + bufb[...]
            src = bufc
        rdma = pltpu.make_async_remote_copy(
            src_ref=src,
            dst_ref=rcv.at[st],
            send_sem=rs_ss.at[st],
            recv_sem=rs_rs.at[st],
            device_id=(right,),
            device_id_type=pl.DeviceIdType.MESH,
        )
        rdma.start()
        rdma.wait()
    ca = pltpu.make_async_copy(rcv.at[N_DEV - 2], bufa, ldma.at[0])
    cb = pltpu.make_async_copy(acc.at[pl.ds(my * CH, CH)], bufb, ldma.at[1])
    ca.start()
    cb.start()
    ca.wait()
    cb.wait()
    out_ref[...] = bufa[...] + bufb[...]

    @functools.partial(pl.run_scoped,
                       second_barrier=pltpu.SemaphoreType.REGULAR)
    def _(second_barrier):
        for nbr in (left, right):
            pl.semaphore_signal(second_barrier, inc=1, device_id=(nbr,),
                                device_id_type=pl.DeviceIdType.MESH)
        pl.semaphore_wait(second_barrier, 2)


def kernel(x, Wq, Wk, Wv, Wo):
    x2d = x.reshape(CH, D)
    out, _, _, _, _ = pl.pallas_call(
        _body,
        out_shape=[
            jax.ShapeDtypeStruct((CH, D), jnp.float32),
            jax.ShapeDtypeStruct((SQ, D), jnp.float32),
            jax.ShapeDtypeStruct((SQ, D), jnp.float32),
            jax.ShapeDtypeStruct((SQ, D), jnp.float32),
            jax.ShapeDtypeStruct((N_DEV - 1, CH, D), jnp.float32),
        ],
        in_specs=[pl.BlockSpec(memory_space=pl.ANY)] * 5
        + [pl.BlockSpec(memory_space=pltpu.VMEM)] * 2,
        out_specs=[pl.BlockSpec(memory_space=pltpu.VMEM)]
        + [pl.BlockSpec(memory_space=pl.ANY)] * 4,
        scratch_shapes=[
            pltpu.VMEM((CH, D), jnp.float32),
            pltpu.VMEM((CH, D), jnp.float32),
            pltpu.VMEM((CH, D), jnp.float32),
            pltpu.VMEM((D, 3 * DH), jnp.float32),
            pltpu.VMEM((SQ, DH), jnp.float32),
            pltpu.VMEM((SQ, DH), jnp.float32),
            pltpu.VMEM((SQ, DH), jnp.float32),
            pltpu.VMEM((SQ, DH), jnp.float32),
            pltpu.SemaphoreType.DMA((4,)),
            pltpu.SemaphoreType.DMA((N_DEV - 1,)),
            pltpu.SemaphoreType.DMA((N_DEV - 1,)),
            pltpu.SemaphoreType.DMA((N_DEV - 1,)),
            pltpu.SemaphoreType.DMA((N_DEV - 1,)),
        ],
        compiler_params=pltpu.CompilerParams(collective_id=0),
    )(x2d, Wq, Wk, Wv, Wo, jnp.asarray(_COS), jnp.asarray(_SIN))
    return out.reshape(1, CH, D)


# device time: 695850 ns/iter; 1.2752x vs baseline; 1.2752x over previous
import functools

import jax
import jax.numpy as jnp
import numpy as np
from jax import lax
from jax.experimental import pallas as pl
from jax.experimental.pallas import tpu as pltpu

N_DEV = 4
SQ = 4096
CH = 1024
D = 1024
H = 8
DH = 128
QB = 256
SCALE = 0.08838834764831843

F32 = jnp.float32
BF16 = jnp.bfloat16


def _rope_tables():
    inv = 1.0 / (10000.0 ** (np.arange(0, DH, 2) / DH))
    pos = np.arange(SQ)[:, None] * inv[None, :]
    cos = np.repeat(np.cos(pos), 2, axis=-1).astype(np.float32)
    sin = np.repeat(np.sin(pos), 2, axis=-1).astype(np.float32)
    return cos, sin


_COS, _SIN = _rope_tables()


def _body(x_ref, wq_ref, wk_ref, wv_ref, wo_ref, cos_ref, sin_ref,
          out_ref, xf, ctx_hbm, acc, rcv,
          bufa, bufb, bufc, bufh, wbuf, cbuf, sbuf, qs, ks, vs, ctxh,
          ldma, ag_ss, ag_rs, rs_ss, rs_rs):
    my = lax.axis_index("i")
    right = lax.rem(my + 1, N_DEV)
    left = lax.rem(my + N_DEV - 1, N_DEV)

    barrier = pltpu.get_barrier_semaphore()
    for nbr in (left, right):
        pl.semaphore_signal(barrier, inc=1, device_id=(nbr,),
                            device_id_type=pl.DeviceIdType.MESH)
    pl.semaphore_wait(barrier, 2)

    cp = pltpu.make_async_copy(x_ref, bufa, ldma.at[0])
    cp.start()
    cp.wait()
    bufh[...] = bufa[...].astype(BF16)
    cp = pltpu.make_async_copy(bufh, xf.at[pl.ds(my * CH, CH)], ldma.at[0])
    cp.start()
    cp.wait()
    for hop in range(N_DEV - 1):
        src_o = lax.rem(my - hop + N_DEV, N_DEV)
        rdma = pltpu.make_async_remote_copy(
            src_ref=xf.at[pl.ds(src_o * CH, CH)],
            dst_ref=xf.at[pl.ds(src_o * CH, CH)],
            send_sem=ag_ss.at[hop],
            recv_sem=ag_rs.at[hop],
            device_id=(right,),
            device_id_type=pl.DeviceIdType.MESH,
        )
        rdma.start()
        rdma.wait()

    lane = lax.broadcasted_iota(jnp.int32, (CH, DH), 1)
    even = (lane % 2) == 0

    def rope(t, c, s):
        t_rot = jnp.where(even, -pltpu.roll(t, DH - 1, 1),
                          pltpu.roll(t, 1, 1))
        return t * c + t_rot * s

    def head_body(h, carry):
        hsl = pl.ds(h * DH, DH)
        cws = []
        for i, w_ref in enumerate((wq_ref, wk_ref, wv_ref)):
            cw = pltpu.make_async_copy(
                w_ref.at[:, hsl], wbuf.at[:, pl.ds(i * DH, DH)], ldma.at[i])
            cw.start()
            cws.append(cw)
        for cw in cws:
            cw.wait()
        wq16 = wbuf[:, 0:DH].astype(BF16)
        wk16 = wbuf[:, DH:2 * DH].astype(BF16)
        wv16 = wbuf[:, 2 * DH:3 * DH].astype(BF16)

        def chunk_body(c, carry):
            csl = pl.ds(c * CH, CH)
            cx = pltpu.make_async_copy(xf.at[csl], bufh, ldma.at[3])
            cc = pltpu.make_async_copy(cos_ref.at[csl], cbuf, ldma.at[0])
            cs = pltpu.make_async_copy(sin_ref.at[csl], sbuf, ldma.at[1])
            cx.start()
            cc.start()
            cs.start()
            cx.wait()
            cc.wait()
            cs.wait()
            xc = bufh[...]
            qc = jnp.dot(xc, wq16, preferred_element_type=F32)
            kc = jnp.dot(xc, wk16, preferred_element_type=F32)
            vc = jnp.dot(xc, wv16, preferred_element_type=F32)
            qs[csl, :] = rope(qc, cbuf[...], sbuf[...]).astype(BF16)
            ks[csl, :] = rope(kc, cbuf[...], sbuf[...]).astype(BF16)
            vs[csl, :] = vc.astype(BF16)
            return carry

        lax.fori_loop(0, N_DEV, chunk_body, 0)

        def qb_body(qb, carry):
            qsl = pl.ds(qb * QB, QB)
            s = lax.dot_general(qs[qsl, :], ks[...], (((1,), (1,)), ((), ())),
                                preferred_element_type=F32) * SCALE
            s = s - jnp.max(s, axis=1, keepdims=True)
            e = jnp.exp(s)
            w = (e / jnp.sum(e, axis=1, keepdims=True)).astype(BF16)
            ctxh[qsl, :] = jnp.dot(w, vs[...],
                                   preferred_element_type=F32).astype(BF16)
            return carry

        lax.fori_loop(0, SQ // QB, qb_body, 0)
        co = pltpu.make_async_copy(ctxh, ctx_hbm.at[:, hsl], ldma.at[3])
        co.start()
        co.wait()
        return carry

    lax.fori_loop(0, H, head_body, 0)

    cw = pltpu.make_async_copy(wo_ref, bufb, ldma.at[0])
    cw.start()
    cw.wait()
    wo16 = bufb[...].astype(BF16)
    for c in range(N_DEV):
        ci = pltpu.make_async_copy(ctx_hbm.at[pl.ds(c * CH, CH)], bufh,
                                   ldma.at[1])
        ci.start()
        ci.wait()
        bufc[...] = jnp.dot(bufh[...], wo16, preferred_element_type=F32)
        co = pltpu.make_async_copy(bufc, acc.at[pl.ds(c * CH, CH)],
                                   ldma.at[2])
        co.start()
        co.wait()

    for st in range(N_DEV - 1):
        c = lax.rem(my - 1 - st + N_DEV, N_DEV)
        if st == 0:
            src = acc.at[pl.ds(c * CH, CH)]
        else:
            ca = pltpu.make_async_copy(rcv.at[st - 1], bufa, ldma.at[0])
            cb = pltpu.make_async_copy(acc.at[pl.ds(c * CH, CH)], bufb,
                                       ldma.at[1])
            ca.start()
            cb.start()
            ca.wait()
            cb.wait()
            bufc[...] = bufa[...] + bufb[...]
            src = bufc
        rdma = pltpu.make_async_remote_copy(
            src_ref=src,
            dst_ref=rcv.at[st],
            send_sem=rs_ss.at[st],
            recv_sem=rs_rs.at[st],
            device_id=(right,),
            device_id_type=pl.DeviceIdType.MESH,
        )
        rdma.start()
        rdma.wait()
    ca = pltpu.make_async_copy(rcv.at[N_DEV - 2], bufa, ldma.at[0])
    cb = pltpu.make_async_copy(acc.at[pl.ds(my * CH, CH)], bufb, ldma.at[1])
    ca.start()
    cb.start()
    ca.wait()
    cb.wait()
    out_ref[...] = bufa[...] + bufb[...]

    @functools.partial(pl.run_scoped,
                       second_barrier=pltpu.SemaphoreType.REGULAR)
    def _(second_barrier):
        for nbr in (left, right):
            pl.semaphore_signal(second_barrier, inc=1, device_id=(nbr,),
                                device_id_type=pl.DeviceIdType.MESH)
        pl.semaphore_wait(second_barrier, 2)


def kernel(x, Wq, Wk, Wv, Wo):
    x2d = x.reshape(CH, D)
    out, _, _, _, _ = pl.pallas_call(
        _body,
        out_shape=[
            jax.ShapeDtypeStruct((CH, D), F32),
            jax.ShapeDtypeStruct((SQ, D), BF16),
            jax.ShapeDtypeStruct((SQ, D), BF16),
            jax.ShapeDtypeStruct((SQ, D), F32),
            jax.ShapeDtypeStruct((N_DEV - 1, CH, D), F32),
        ],
        in_specs=[pl.BlockSpec(memory_space=pl.ANY)] * 7,
        out_specs=[pl.BlockSpec(memory_space=pltpu.VMEM)]
        + [pl.BlockSpec(memory_space=pl.ANY)] * 4,
        scratch_shapes=[
            pltpu.VMEM((CH, D), F32),
            pltpu.VMEM((CH, D), F32),
            pltpu.VMEM((CH, D), F32),
            pltpu.VMEM((CH, D), BF16),
            pltpu.VMEM((D, 3 * DH), F32),
            pltpu.VMEM((CH, DH), F32),
            pltpu.VMEM((CH, DH), F32),
            pltpu.VMEM((SQ, DH), BF16),
            pltpu.VMEM((SQ, DH), BF16),
            pltpu.VMEM((SQ, DH), BF16),
            pltpu.VMEM((SQ, DH), BF16),
            pltpu.SemaphoreType.DMA((4,)),
            pltpu.SemaphoreType.DMA((N_DEV - 1,)),
            pltpu.SemaphoreType.DMA((N_DEV - 1,)),
            pltpu.SemaphoreType.DMA((N_DEV - 1,)),
            pltpu.SemaphoreType.DMA((N_DEV - 1,)),
        ],
        compiler_params=pltpu.CompilerParams(collective_id=0),
    )(x2d, Wq, Wk, Wv, Wo, jnp.asarray(_COS), jnp.asarray(_SIN))
    return out.reshape(1, CH, D)


# device time: 571461 ns/iter; 1.5528x vs baseline; 1.2177x over previous
import functools

import jax
import jax.numpy as jnp
import numpy as np
from jax import lax
from jax.experimental import pallas as pl
from jax.experimental.pallas import tpu as pltpu

N_DEV = 4
SQ = 4096
CH = 1024
D = 1024
H = 8
DH = 128
QB = 256
SCALE = 0.08838834764831843

F32 = jnp.float32
BF16 = jnp.bfloat16


def _rope_tables():
    inv = 1.0 / (10000.0 ** (np.arange(0, DH, 2) / DH))
    pos = np.arange(SQ)[:, None] * inv[None, :]
    cos = np.repeat(np.cos(pos), 2, axis=-1).astype(np.float32)
    sin = np.repeat(np.sin(pos), 2, axis=-1).astype(np.float32)
    return cos, sin


_COS, _SIN = _rope_tables()


def _body(x_ref, wq_ref, wk_ref, wv_ref, wo_ref, cos_ref, sin_ref,
          out_ref, ctx_hbm, acc, rcv,
          xf, fb1, hb1, hb2, cosb, sinb, qs, ks, vs, ctxh,
          ldma, ag_ss, ag_rs, rs_ss, rs_rs):
    my = lax.axis_index("i")
    right = lax.rem(my + 1, N_DEV)
    left = lax.rem(my + N_DEV - 1, N_DEV)

    barrier = pltpu.get_barrier_semaphore()
    for nbr in (left, right):
        pl.semaphore_signal(barrier, inc=1, device_id=(nbr,),
                            device_id_type=pl.DeviceIdType.MESH)
    pl.semaphore_wait(barrier, 2)

    cp = pltpu.make_async_copy(x_ref, fb1, ldma.at[0])
    cp.start()
    cp.wait()
    xf[pl.ds(my * CH, CH), :] = fb1[...].astype(BF16)
    for hop in range(N_DEV - 1):
        src_o = lax.rem(my - hop + N_DEV, N_DEV)
        rdma = pltpu.make_async_remote_copy(
            src_ref=xf.at[pl.ds(src_o * CH, CH)],
            dst_ref=xf.at[pl.ds(src_o * CH, CH)],
            send_sem=ag_ss.at[hop],
            recv_sem=ag_rs.at[hop],
            device_id=(right,),
            device_id_type=pl.DeviceIdType.MESH,
        )
        rdma.start()
        rdma.wait()

    for c in range(N_DEV):
        csl = pl.ds(c * CH, CH)
        cc = pltpu.make_async_copy(cos_ref.at[csl], fb1.at[:, 0:DH],
                                   ldma.at[0])
        cs = pltpu.make_async_copy(sin_ref.at[csl], fb1.at[:, DH:2 * DH],
                                   ldma.at[1])
        cc.start()
        cs.start()
        cc.wait()
        cs.wait()
        cosb[csl, :] = fb1[:, 0:DH].astype(BF16)
        sinb[csl, :] = fb1[:, DH:2 * DH].astype(BF16)

    lane = lax.broadcasted_iota(jnp.int32, (CH, DH), 1)
    even = (lane % 2) == 0

    def rope(t, c, s):
        t_rot = jnp.where(even, -pltpu.roll(t, DH - 1, 1),
                          pltpu.roll(t, 1, 1))
        return t * c + t_rot * s

    def head_body(h, carry):
        hsl = pl.ds(h * DH, DH)
        cws = []
        for i, w_ref in enumerate((wq_ref, wk_ref, wv_ref)):
            cw = pltpu.make_async_copy(
                w_ref.at[:, hsl], fb1.at[:, pl.ds(i * DH, DH)], ldma.at[i])
            cw.start()
            cws.append(cw)
        for cw in cws:
            cw.wait()
        wq16 = fb1[:, 0:DH].astype(BF16)
        wk16 = fb1[:, DH:2 * DH].astype(BF16)
        wv16 = fb1[:, 2 * DH:3 * DH].astype(BF16)

        def chunk_body(c, carry):
            csl = pl.ds(c * CH, CH)
            xc = xf[csl, :]
            qc = jnp.dot(xc, wq16, preferred_element_type=F32)
            kc = jnp.dot(xc, wk16, preferred_element_type=F32)
            vc = jnp.dot(xc, wv16, preferred_element_type=F32)
            qs[csl, :] = rope(qc, cosb[csl, :], sinb[csl, :]).astype(BF16)
            ks[csl, :] = rope(kc, cosb[csl, :], sinb[csl, :]).astype(BF16)
            vs[csl, :] = vc.astype(BF16)
            return carry

        lax.fori_loop(0, N_DEV, chunk_body, 0)

        def qb_body(qb, carry):
            qsl = pl.ds(qb * QB, QB)
            s = lax.dot_general(qs[qsl, :], ks[...], (((1,), (1,)), ((), ())),
                                preferred_element_type=F32) * SCALE
            s = s - jnp.max(s, axis=1, keepdims=True)
            e = jnp.exp(s)
            w = (e / jnp.sum(e, axis=1, keepdims=True)).astype(BF16)
            ctxh[qsl, :] = jnp.dot(w, vs[...],
                                   preferred_element_type=F32).astype(BF16)
            return carry

        lax.fori_loop(0, SQ // QB, qb_body, 0)
        co = pltpu.make_async_copy(ctxh, ctx_hbm.at[:, hsl], ldma.at[3])
        co.start()
        co.wait()
        return carry

    lax.fori_loop(0, H, head_body, 0)

    cw = pltpu.make_async_copy(wo_ref, fb1, ldma.at[0])
    cw.start()
    cw.wait()
    wo16 = fb1[...].astype(BF16)
    for c in range(N_DEV):
        ci = pltpu.make_async_copy(ctx_hbm.at[pl.ds(c * CH, CH)], hb1,
                                   ldma.at[1])
        ci.start()
        ci.wait()
        hb2[...] = jnp.dot(hb1[...], wo16,
                           preferred_element_type=F32).astype(BF16)
        co = pltpu.make_async_copy(hb2, acc.at[pl.ds(c * CH, CH)],
                                   ldma.at[2])
        co.start()
        co.wait()

    for st in range(N_DEV - 1):
        c = lax.rem(my - 1 - st + N_DEV, N_DEV)
        if st == 0:
            src = acc.at[pl.ds(c * CH, CH)]
        else:
            ca = pltpu.make_async_copy(rcv.at[st - 1], hb1, ldma.at[0])
            cb = pltpu.make_async_copy(acc.at[pl.ds(c * CH, CH)], hb2,
                                       ldma.at[1])
            ca.start()
            cb.start()
            ca.wait()
            cb.wait()
            hb2[...] = (hb1[...].astype(F32)
                        + hb2[...].astype(F32)).astype(BF16)
            src = hb2
        rdma = pltpu.make_async_remote_copy(
            src_ref=src,
            dst_ref=rcv.at[st],
            send_sem=rs_ss.at[st],
            recv_sem=rs_rs.at[st],
            device_id=(right,),
            device_id_type=pl.DeviceIdType.MESH,
        )
        rdma.start()
        rdma.wait()
    ca = pltpu.make_async_copy(rcv.at[N_DEV - 2], hb1, ldma.at[0])
    cb = pltpu.make_async_copy(acc.at[pl.ds(my * CH, CH)], hb2, ldma.at[1])
    ca.start()
    cb.start()
    ca.wait()
    cb.wait()
    out_ref[...] = hb1[...].astype(F32) + hb2[...].astype(F32)

    @functools.partial(pl.run_scoped,
                       second_barrier=pltpu.SemaphoreType.REGULAR)
    def _(second_barrier):
        for nbr in (left, right):
            pl.semaphore_signal(second_barrier, inc=1, device_id=(nbr,),
                                device_id_type=pl.DeviceIdType.MESH)
        pl.semaphore_wait(second_barrier, 2)


def kernel(x, Wq, Wk, Wv, Wo):
    x2d = x.reshape(CH, D)
    out, _, _, _ = pl.pallas_call(
        _body,
        out_shape=[
            jax.ShapeDtypeStruct((CH, D), F32),
            jax.ShapeDtypeStruct((SQ, D), BF16),
            jax.ShapeDtypeStruct((SQ, D), BF16),
            jax.ShapeDtypeStruct((N_DEV - 1, CH, D), BF16),
        ],
        in_specs=[pl.BlockSpec(memory_space=pl.ANY)] * 7,
        out_specs=[pl.BlockSpec(memory_space=pltpu.VMEM)]
        + [pl.BlockSpec(memory_space=pl.ANY)] * 3,
        scratch_shapes=[
            pltpu.VMEM((SQ, D), BF16),
            pltpu.VMEM((CH, D), F32),
            pltpu.VMEM((CH, D), BF16),
            pltpu.VMEM((CH, D), BF16),
            pltpu.VMEM((SQ, DH), BF16),
            pltpu.VMEM((SQ, DH), BF16),
            pltpu.VMEM((SQ, DH), BF16),
            pltpu.VMEM((SQ, DH), BF16),
            pltpu.VMEM((SQ, DH), BF16),
            pltpu.VMEM((SQ, DH), BF16),
            pltpu.SemaphoreType.DMA((4,)),
            pltpu.SemaphoreType.DMA((N_DEV - 1,)),
            pltpu.SemaphoreType.DMA((N_DEV - 1,)),
            pltpu.SemaphoreType.DMA((N_DEV - 1,)),
            pltpu.SemaphoreType.DMA((N_DEV - 1,)),
        ],
        compiler_params=pltpu.CompilerParams(collective_id=0),
    )(x2d, Wq, Wk, Wv, Wo, jnp.asarray(_COS), jnp.asarray(_SIN))
    return out.reshape(1, CH, D)


# device time: 446955 ns/iter; 1.9853x vs baseline; 1.2786x over previous
import functools

import jax
import jax.numpy as jnp
import numpy as np
from jax import lax
from jax.experimental import pallas as pl
from jax.experimental.pallas import tpu as pltpu

N_DEV = 4
SQ = 4096
CH = 1024
D = 1024
H = 8
DH = 128
QB = 256
SCALE = 0.08838834764831843

F32 = jnp.float32
BF16 = jnp.bfloat16


def _rope_tables():
    inv = 1.0 / (10000.0 ** (np.arange(0, DH, 2) / DH))
    pos = np.arange(SQ)[:, None] * inv[None, :]
    cos = np.repeat(np.cos(pos), 2, axis=-1).astype(np.float32)
    sin = np.repeat(np.sin(pos), 2, axis=-1).astype(np.float32)
    return cos, sin


_COS, _SIN = _rope_tables()


def _body(x_ref, wq_ref, wk_ref, wv_ref, wo_ref, cos_ref, sin_ref,
          out_ref, ctx_hbm, acc, rcv,
          xf, fb1, hb1, hb2, cosb, sinb, qs, ks, vs, ctxh,
          ldma, ag_ss, ag_rs, rs_ss, rs_rs):
    my = lax.axis_index("i")
    right = lax.rem(my + 1, N_DEV)
    left = lax.rem(my + N_DEV - 1, N_DEV)

    barrier = pltpu.get_barrier_semaphore()
    for nbr in (left, right):
        pl.semaphore_signal(barrier, inc=1, device_id=(nbr,),
                            device_id_type=pl.DeviceIdType.MESH)
    pl.semaphore_wait(barrier, 2)

    cp = pltpu.make_async_copy(x_ref, fb1, ldma.at[0])
    cp.start()
    cp.wait()
    xf[pl.ds(my * CH, CH), :] = fb1[...].astype(BF16)
    for hop in range(N_DEV - 1):
        src_o = lax.rem(my - hop + N_DEV, N_DEV)
        rdma = pltpu.make_async_remote_copy(
            src_ref=xf.at[pl.ds(src_o * CH, CH)],
            dst_ref=xf.at[pl.ds(src_o * CH, CH)],
            send_sem=ag_ss.at[hop],
            recv_sem=ag_rs.at[hop],
            device_id=(right,),
            device_id_type=pl.DeviceIdType.MESH,
        )
        rdma.start()
        rdma.wait()

    for c in range(N_DEV):
        csl = pl.ds(c * CH, CH)
        cc = pltpu.make_async_copy(cos_ref.at[csl], fb1.at[:, 0:DH],
                                   ldma.at[0])
        cs = pltpu.make_async_copy(sin_ref.at[csl], fb1.at[:, DH:2 * DH],
                                   ldma.at[1])
        cc.start()
        cs.start()
        cc.wait()
        cs.wait()
        cosb[csl, :] = fb1[:, 0:DH].astype(BF16)
        sinb[csl, :] = fb1[:, DH:2 * DH].astype(BF16)

    lane = lax.broadcasted_iota(jnp.int32, (CH, DH), 1)
    even = (lane % 2) == 0

    def rope(t, c, s):
        t_rot = jnp.where(even, -pltpu.roll(t, DH - 1, 1),
                          pltpu.roll(t, 1, 1))
        return t * c + t_rot * s

    def head_body(h, carry):
        hsl = pl.ds(h * DH, DH)
        cws = []
        for i, w_ref in enumerate((wq_ref, wk_ref, wv_ref)):
            cw = pltpu.make_async_copy(
                w_ref.at[:, hsl], fb1.at[:, pl.ds(i * DH, DH)], ldma.at[i])
            cw.start()
            cws.append(cw)
        for cw in cws:
            cw.wait()
        wq16 = fb1[:, 0:DH].astype(BF16)
        wk16 = fb1[:, DH:2 * DH].astype(BF16)
        wv16 = fb1[:, 2 * DH:3 * DH].astype(BF16)

        def chunk_body(c, carry):
            csl = pl.ds(c * CH, CH)
            xc = xf[csl, :]
            qc = jnp.dot(xc, wq16, preferred_element_type=F32)
            kc = jnp.dot(xc, wk16, preferred_element_type=F32)
            vc = jnp.dot(xc, wv16, preferred_element_type=F32)
            qs[csl, :] = (rope(qc, cosb[csl, :], sinb[csl, :])
                          * SCALE).astype(BF16)
            ks[csl, :] = rope(kc, cosb[csl, :], sinb[csl, :]).astype(BF16)
            vs[csl, :] = vc.astype(BF16)
            return carry

        lax.fori_loop(0, N_DEV, chunk_body, 0)

        def qb_body(qb, carry):
            qsl = pl.ds(qb * QB, QB)
            s = lax.dot_general(qs[qsl, :], ks[...], (((1,), (1,)), ((), ())),
                                preferred_element_type=F32)
            e = jnp.exp(s)
            denom = jnp.sum(e, axis=1, keepdims=True)
            ctxu = jnp.dot(e.astype(BF16), vs[...],
                           preferred_element_type=F32)
            ctxh[qsl, :] = (ctxu / denom).astype(BF16)
            return carry

        lax.fori_loop(0, SQ // QB, qb_body, 0)
        co = pltpu.make_async_copy(ctxh, ctx_hbm.at[:, hsl], ldma.at[3])
        co.start()
        co.wait()
        return carry

    lax.fori_loop(0, H, head_body, 0)

    cw = pltpu.make_async_copy(wo_ref, fb1, ldma.at[0])
    cw.start()
    cw.wait()
    wo16 = fb1[...].astype(BF16)
    for c in range(N_DEV):
        ci = pltpu.make_async_copy(ctx_hbm.at[pl.ds(c * CH, CH)], hb1,
                                   ldma.at[1])
        ci.start()
        ci.wait()
        hb2[...] = jnp.dot(hb1[...], wo16,
                           preferred_element_type=F32).astype(BF16)
        co = pltpu.make_async_copy(hb2, acc.at[pl.ds(c * CH, CH)],
                                   ldma.at[2])
        co.start()
        co.wait()

    for st in range(N_DEV - 1):
        c = lax.rem(my - 1 - st + N_DEV, N_DEV)
        if st == 0:
            src = acc.at[pl.ds(c * CH, CH)]
        else:
            ca = pltpu.make_async_copy(rcv.at[st - 1], hb1, ldma.at[0])
            cb = pltpu.make_async_copy(acc.at[pl.ds(c * CH, CH)], hb2,
                                       ldma.at[1])
            ca.start()
            cb.start()
            ca.wait()
            cb.wait()
            hb2[...] = (hb1[...].astype(F32)
                        + hb2[...].astype(F32)).astype(BF16)
            src = hb2
        rdma = pltpu.make_async_remote_copy(
            src_ref=src,
            dst_ref=rcv.at[st],
            send_sem=rs_ss.at[st],
            recv_sem=rs_rs.at[st],
            device_id=(right,),
            device_id_type=pl.DeviceIdType.MESH,
        )
        rdma.start()
        rdma.wait()
    ca = pltpu.make_async_copy(rcv.at[N_DEV - 2], hb1, ldma.at[0])
    cb = pltpu.make_async_copy(acc.at[pl.ds(my * CH, CH)], hb2, ldma.at[1])
    ca.start()
    cb.start()
    ca.wait()
    cb.wait()
    out_ref[...] = hb1[...].astype(F32) + hb2[...].astype(F32)

    @functools.partial(pl.run_scoped,
                       second_barrier=pltpu.SemaphoreType.REGULAR)
    def _(second_barrier):
        for nbr in (left, right):
            pl.semaphore_signal(second_barrier, inc=1, device_id=(nbr,),
                                device_id_type=pl.DeviceIdType.MESH)
        pl.semaphore_wait(second_barrier, 2)


def kernel(x, Wq, Wk, Wv, Wo):
    x2d = x.reshape(CH, D)
    out, _, _, _ = pl.pallas_call(
        _body,
        out_shape=[
            jax.ShapeDtypeStruct((CH, D), F32),
            jax.ShapeDtypeStruct((SQ, D), BF16),
            jax.ShapeDtypeStruct((SQ, D), BF16),
            jax.ShapeDtypeStruct((N_DEV - 1, CH, D), BF16),
        ],
        in_specs=[pl.BlockSpec(memory_space=pl.ANY)] * 7,
        out_specs=[pl.BlockSpec(memory_space=pltpu.VMEM)]
        + [pl.BlockSpec(memory_space=pl.ANY)] * 3,
        scratch_shapes=[
            pltpu.VMEM((SQ, D), BF16),
            pltpu.VMEM((CH, D), F32),
            pltpu.VMEM((CH, D), BF16),
            pltpu.VMEM((CH, D), BF16),
            pltpu.VMEM((SQ, DH), BF16),
            pltpu.VMEM((SQ, DH), BF16),
            pltpu.VMEM((SQ, DH), BF16),
            pltpu.VMEM((SQ, DH), BF16),
            pltpu.VMEM((SQ, DH), BF16),
            pltpu.VMEM((SQ, DH), BF16),
            pltpu.SemaphoreType.DMA((4,)),
            pltpu.SemaphoreType.DMA((N_DEV - 1,)),
            pltpu.SemaphoreType.DMA((N_DEV - 1,)),
            pltpu.SemaphoreType.DMA((N_DEV - 1,)),
            pltpu.SemaphoreType.DMA((N_DEV - 1,)),
        ],
        compiler_params=pltpu.CompilerParams(collective_id=0),
    )(x2d, Wq, Wk, Wv, Wo, jnp.asarray(_COS), jnp.asarray(_SIN))
    return out.reshape(1, CH, D)


# device time: 416260 ns/iter; 2.1317x vs baseline; 1.0737x over previous
import functools

import jax
import jax.numpy as jnp
import numpy as np
from jax import lax
from jax.experimental import pallas as pl
from jax.experimental.pallas import tpu as pltpu

N_DEV = 4
SQ = 4096
CH = 1024
D = 1024
H = 8
DH = 128
QB = 512
SCALE = 0.08838834764831843

F32 = jnp.float32
BF16 = jnp.bfloat16


def _rope_tables():
    inv = 1.0 / (10000.0 ** (np.arange(0, DH, 2) / DH))
    pos = np.arange(SQ)[:, None] * inv[None, :]
    cos = np.repeat(np.cos(pos), 2, axis=-1).astype(np.float32)
    sin = np.repeat(np.sin(pos), 2, axis=-1).astype(np.float32)
    return cos, sin


_COS, _SIN = _rope_tables()


def _body(x_ref, wq_ref, wk_ref, wv_ref, wo_ref, cos_ref, sin_ref,
          out_ref, ctx_hbm, rcv,
          xf, fb1, hb1, hb2, cosb, sinb, qs, ks, vs, ctxh,
          ldma, ag_ss, ag_rs, rs_ss, rs_rs):
    my = lax.axis_index("i")
    right = lax.rem(my + 1, N_DEV)
    left = lax.rem(my + N_DEV - 1, N_DEV)

    barrier = pltpu.get_barrier_semaphore()
    for nbr in (left, right):
        pl.semaphore_signal(barrier, inc=1, device_id=(nbr,),
                            device_id_type=pl.DeviceIdType.MESH)
    pl.semaphore_wait(barrier, 2)

    cp = pltpu.make_async_copy(x_ref, fb1, ldma.at[0])
    cp.start()
    cp.wait()
    xf[pl.ds(my * CH, CH), :] = fb1[...].astype(BF16)

    def _ag_hop(hop):
        src_o = lax.rem(my - hop + N_DEV, N_DEV)
        return pltpu.make_async_remote_copy(
            src_ref=xf.at[pl.ds(src_o * CH, CH)],
            dst_ref=xf.at[pl.ds(src_o * CH, CH)],
            send_sem=ag_ss.at[hop],
            recv_sem=ag_rs.at[hop],
            device_id=(right,),
            device_id_type=pl.DeviceIdType.MESH,
        )

    rdma = _ag_hop(0)
    rdma.start()
    for c in range(N_DEV):
        csl = pl.ds(c * CH, CH)
        cc = pltpu.make_async_copy(cos_ref.at[csl], fb1.at[:, 0:DH],
                                   ldma.at[1])
        cs = pltpu.make_async_copy(sin_ref.at[csl], fb1.at[:, DH:2 * DH],
                                   ldma.at[2])
        cc.start()
        cs.start()
        cc.wait()
        cs.wait()
        cosb[csl, :] = fb1[:, 0:DH].astype(BF16)
        sinb[csl, :] = fb1[:, DH:2 * DH].astype(BF16)
    rdma.wait()
    for hop in (1, 2):
        rdma = _ag_hop(hop)
        rdma.start()
        rdma.wait()

    lane = lax.broadcasted_iota(jnp.int32, (CH, DH), 1)
    even = (lane % 2) == 0

    def rope(t, c, s):
        t_rot = jnp.where(even, -pltpu.roll(t, DH - 1, 1),
                          pltpu.roll(t, 1, 1))
        return t * c + t_rot * s

    def head_body(h, carry):
        hsl = pl.ds(h * DH, DH)
        cws = []
        for i, w_ref in enumerate((wq_ref, wk_ref, wv_ref)):
            cw = pltpu.make_async_copy(
                w_ref.at[:, hsl], fb1.at[:, pl.ds(i * DH, DH)], ldma.at[i])
            cw.start()
            cws.append(cw)
        for cw in cws:
            cw.wait()
        wq16 = fb1[:, 0:DH].astype(BF16)
        wk16 = fb1[:, DH:2 * DH].astype(BF16)
        wv16 = fb1[:, 2 * DH:3 * DH].astype(BF16)

        def chunk_body(c, carry):
            csl = pl.ds(c * CH, CH)
            xc = xf[csl, :]
            qc = jnp.dot(xc, wq16, preferred_element_type=F32)
            kc = jnp.dot(xc, wk16, preferred_element_type=F32)
            vc = jnp.dot(xc, wv16, preferred_element_type=F32)
            qs[csl, :] = (rope(qc, cosb[csl, :], sinb[csl, :])
                          * SCALE).astype(BF16)
            ks[csl, :] = rope(kc, cosb[csl, :], sinb[csl, :]).astype(BF16)
            vs[csl, :] = vc.astype(BF16)
            return carry

        lax.fori_loop(0, N_DEV, chunk_body, 0)

        def qb_body(qb, carry):
            qsl = pl.ds(qb * QB, QB)
            s = lax.dot_general(qs[qsl, :], ks[...], (((1,), (1,)), ((), ())),
                                preferred_element_type=F32)
            e = jnp.exp(s)
            denom = jnp.sum(e, axis=1, keepdims=True)
            ctxu = jnp.dot(e.astype(BF16), vs[...],
                           preferred_element_type=F32)
            ctxh[qsl, :] = (ctxu / denom).astype(BF16)
            return carry

        lax.fori_loop(0, SQ // QB, qb_body, 0)
        co = pltpu.make_async_copy(ctxh, ctx_hbm.at[:, hsl], ldma.at[3])
        co.start()
        co.wait()
        return carry

    lax.fori_loop(0, H, head_body, 0)

    cw = pltpu.make_async_copy(wo_ref, fb1, ldma.at[0])
    cw.start()
    cw.wait()
    wo16 = fb1[...].astype(BF16)

    def _load_ctx(c):
        ci = pltpu.make_async_copy(ctx_hbm.at[pl.ds(c * CH, CH)], hb1,
                                   ldma.at[1])
        ci.start()
        ci.wait()
        return jnp.dot(hb1[...], wo16, preferred_element_type=F32)

    def _rs_hop(st, src):
        return pltpu.make_async_remote_copy(
            src_ref=src,
            dst_ref=rcv.at[st],
            send_sem=rs_ss.at[st],
            recv_sem=rs_rs.at[st],
            device_id=(right,),
            device_id_type=pl.DeviceIdType.MESH,
        )

    hb2[...] = _load_ctx(lax.rem(my - 1 + N_DEV, N_DEV)).astype(BF16)
    rdma = _rs_hop(0, hb2)
    rdma.start()
    for st in (1, 2):
        d = _load_ctx(lax.rem(my - 1 - st + N_DEV, N_DEV))
        rdma.wait()
        cr = pltpu.make_async_copy(rcv.at[st - 1], hb2, ldma.at[2])
        cr.start()
        cr.wait()
        hb2[...] = (hb2[...].astype(F32) + d).astype(BF16)
        rdma = _rs_hop(st, hb2)
        rdma.start()
    d = _load_ctx(my)
    rdma.wait()
    cr = pltpu.make_async_copy(rcv.at[N_DEV - 2], hb2, ldma.at[2])
    cr.start()
    cr.wait()
    out_ref[...] = hb2[...].astype(F32) + d

    @functools.partial(pl.run_scoped,
                       second_barrier=pltpu.SemaphoreType.REGULAR)
    def _(second_barrier):
        for nbr in (left, right):
            pl.semaphore_signal(second_barrier, inc=1, device_id=(nbr,),
                                device_id_type=pl.DeviceIdType.MESH)
        pl.semaphore_wait(second_barrier, 2)


def kernel(x, Wq, Wk, Wv, Wo):
    x2d = x.reshape(CH, D)
    out, _, _ = pl.pallas_call(
        _body,
        out_shape=[
            jax.ShapeDtypeStruct((CH, D), F32),
            jax.ShapeDtypeStruct((SQ, D), BF16),
            jax.ShapeDtypeStruct((N_DEV - 1, CH, D), BF16),
        ],
        in_specs=[pl.BlockSpec(memory_space=pl.ANY)] * 7,
        out_specs=[pl.BlockSpec(memory_space=pltpu.VMEM)]
        + [pl.BlockSpec(memory_space=pl.ANY)] * 2,
        scratch_shapes=[
            pltpu.VMEM((SQ, D), BF16),
            pltpu.VMEM((CH, D), F32),
            pltpu.VMEM((CH, D), BF16),
            pltpu.VMEM((CH, D), BF16),
            pltpu.VMEM((SQ, DH), BF16),
            pltpu.VMEM((SQ, DH), BF16),
            pltpu.VMEM((SQ, DH), BF16),
            pltpu.VMEM((SQ, DH), BF16),
            pltpu.VMEM((SQ, DH), BF16),
            pltpu.VMEM((SQ, DH), BF16),
            pltpu.SemaphoreType.DMA((4,)),
            pltpu.SemaphoreType.DMA((N_DEV - 1,)),
            pltpu.SemaphoreType.DMA((N_DEV - 1,)),
            pltpu.SemaphoreType.DMA((N_DEV - 1,)),
            pltpu.SemaphoreType.DMA((N_DEV - 1,)),
        ],
        compiler_params=pltpu.CompilerParams(collective_id=0),
    )(x2d, Wq, Wk, Wv, Wo, jnp.asarray(_COS), jnp.asarray(_SIN))
    return out.reshape(1, CH, D)


# device time: 383797 ns/iter; 2.3120x vs baseline; 1.0846x over previous
import functools

import jax
import jax.numpy as jnp
import numpy as np
from jax import lax
from jax.experimental import pallas as pl
from jax.experimental.pallas import tpu as pltpu

N_DEV = 4
SQ = 4096
CH = 1024
D = 1024
H = 8
DH = 128
QB = 512
SCALE = 0.08838834764831843

F32 = jnp.float32
BF16 = jnp.bfloat16


def _rope_tables():
    inv = 1.0 / (10000.0 ** (np.arange(0, DH, 2) / DH))
    pos = np.arange(SQ)[:, None] * inv[None, :]
    cos = np.repeat(np.cos(pos), 2, axis=-1).astype(np.float32)
    sin = np.repeat(np.sin(pos), 2, axis=-1).astype(np.float32)
    return cos, sin


_COS, _SIN = _rope_tables()


def _body(x_ref, wq_ref, wk_ref, wv_ref, wo_ref, cos_ref, sin_ref,
          out_ref, ctx_hbm, rcv,
          xf, fb1, hb1, hb2, cosb, sinb, qs, ks, vs, ctxh,
          ldma, ag_ss, ag_rs, al_ss, al_rs, rs_ss, rs_rs):
    my = lax.axis_index("i")
    right = lax.rem(my + 1, N_DEV)
    left = lax.rem(my + N_DEV - 1, N_DEV)

    barrier = pltpu.get_barrier_semaphore()
    for nbr in (left, right):
        pl.semaphore_signal(barrier, inc=1, device_id=(nbr,),
                            device_id_type=pl.DeviceIdType.MESH)
    pl.semaphore_wait(barrier, 2)

    cp = pltpu.make_async_copy(x_ref, fb1, ldma.at[0])
    cp.start()
    cp.wait()
    xf[pl.ds(my * CH, CH), :] = fb1[...].astype(BF16)

    HC = D // 2

    def _ag_hop(hop):
        o_r = lax.rem(my - hop + N_DEV, N_DEV)
        o_l = lax.rem(my + hop, N_DEV)
        r = pltpu.make_async_remote_copy(
            src_ref=xf.at[pl.ds(o_r * CH, CH), pl.ds(0, HC)],
            dst_ref=xf.at[pl.ds(o_r * CH, CH), pl.ds(0, HC)],
            send_sem=ag_ss.at[hop],
            recv_sem=ag_rs.at[hop],
            device_id=(right,),
            device_id_type=pl.DeviceIdType.MESH,
        )
        l = pltpu.make_async_remote_copy(
            src_ref=xf.at[pl.ds(o_l * CH, CH), pl.ds(HC, HC)],
            dst_ref=xf.at[pl.ds(o_l * CH, CH), pl.ds(HC, HC)],
            send_sem=al_ss.at[hop],
            recv_sem=al_rs.at[hop],
            device_id=(left,),
            device_id_type=pl.DeviceIdType.MESH,
        )
        return r, l

    rdma_r, rdma_l = _ag_hop(0)
    rdma_r.start()
    rdma_l.start()
    for c in range(N_DEV):
        csl = pl.ds(c * CH, CH)
        cc = pltpu.make_async_copy(cos_ref.at[csl], fb1.at[:, 0:DH],
                                   ldma.at[1])
        cs = pltpu.make_async_copy(sin_ref.at[csl], fb1.at[:, DH:2 * DH],
                                   ldma.at[2])
        cc.start()
        cs.start()
        cc.wait()
        cs.wait()
        cosb[csl, :] = fb1[:, 0:DH].astype(BF16)
        sinb[csl, :] = fb1[:, DH:2 * DH].astype(BF16)
    rdma_r.wait()
    rdma_l.wait()
    for hop in (1, 2):
        rdma_r, rdma_l = _ag_hop(hop)
        rdma_r.start()
        rdma_l.start()
        rdma_r.wait()
        rdma_l.wait()

    lane = lax.broadcasted_iota(jnp.int32, (CH, DH), 1)
    even = (lane % 2) == 0

    def rope(t, c, s):
        t_rot = jnp.where(even, -pltpu.roll(t, DH - 1, 1),
                          pltpu.roll(t, 1, 1))
        return t * c + t_rot * s

    def head_body(h, carry):
        hsl = pl.ds(h * DH, DH)
        cws = []
        for i, w_ref in enumerate((wq_ref, wk_ref, wv_ref)):
            cw = pltpu.make_async_copy(
                w_ref.at[:, hsl], fb1.at[:, pl.ds(i * DH, DH)], ldma.at[i])
            cw.start()
            cws.append(cw)
        for cw in cws:
            cw.wait()
        wq16 = fb1[:, 0:DH].astype(BF16)
        wk16 = fb1[:, DH:2 * DH].astype(BF16)
        wv16 = fb1[:, 2 * DH:3 * DH].astype(BF16)

        def chunk_body(c, carry):
            csl = pl.ds(c * CH, CH)
            xc = xf[csl, :]
            qc = jnp.dot(xc, wq16, preferred_element_type=F32)
            kc = jnp.dot(xc, wk16, preferred_element_type=F32)
            vc = jnp.dot(xc, wv16, preferred_element_type=F32)
            qs[csl, :] = (rope(qc, cosb[csl, :], sinb[csl, :])
                          * SCALE).astype(BF16)
            ks[csl, :] = rope(kc, cosb[csl, :], sinb[csl, :]).astype(BF16)
            vs[csl, :] = vc.astype(BF16)
            return carry

        lax.fori_loop(0, N_DEV, chunk_body, 0)

        def qb_body(qb, carry):
            qsl = pl.ds(qb * QB, QB)
            s = lax.dot_general(qs[qsl, :], ks[...], (((1,), (1,)), ((), ())),
                                preferred_element_type=F32)
            e = jnp.exp(s)
            denom = jnp.sum(e, axis=1, keepdims=True)
            ctxu = jnp.dot(e.astype(BF16), vs[...],
                           preferred_element_type=F32)
            ctxh[qsl, :] = (ctxu / denom).astype(BF16)
            return carry

        lax.fori_loop(0, SQ // QB, qb_body, 0)
        co = pltpu.make_async_copy(ctxh, ctx_hbm.at[:, hsl], ldma.at[3])
        co.start()
        co.wait()
        return carry

    lax.fori_loop(0, H, head_body, 0)

    cw = pltpu.make_async_copy(wo_ref, fb1, ldma.at[0])
    cw.start()
    cw.wait()
    wo16 = fb1[...].astype(BF16)

    def _load_ctx(c):
        ci = pltpu.make_async_copy(ctx_hbm.at[pl.ds(c * CH, CH)], hb1,
                                   ldma.at[1])
        ci.start()
        ci.wait()
        return jnp.dot(hb1[...], wo16, preferred_element_type=F32)

    def _rs_hop(st, src):
        return pltpu.make_async_remote_copy(
            src_ref=src,
            dst_ref=rcv.at[st],
            send_sem=rs_ss.at[st],
            recv_sem=rs_rs.at[st],
            device_id=(right,),
            device_id_type=pl.DeviceIdType.MESH,
        )

    hb2[...] = _load_ctx(lax.rem(my - 1 + N_DEV, N_DEV)).astype(BF16)
    rdma = _rs_hop(0, hb2)
    rdma.start()
    for st in (1, 2):
        d = _load_ctx(lax.rem(my - 1 - st + N_DEV, N_DEV))
        rdma.wait()
        cr = pltpu.make_async_copy(rcv.at[st - 1], hb2, ldma.at[2])
        cr.start()
        cr.wait()
        hb2[...] = (hb2[...].astype(F32) + d).astype(BF16)
        rdma = _rs_hop(st, hb2)
        rdma.start()
    d = _load_ctx(my)
    rdma.wait()
    cr = pltpu.make_async_copy(rcv.at[N_DEV - 2], hb2, ldma.at[2])
    cr.start()
    cr.wait()
    out_ref[...] = hb2[...].astype(F32) + d

    @functools.partial(pl.run_scoped,
                       second_barrier=pltpu.SemaphoreType.REGULAR)
    def _(second_barrier):
        for nbr in (left, right):
            pl.semaphore_signal(second_barrier, inc=1, device_id=(nbr,),
                                device_id_type=pl.DeviceIdType.MESH)
        pl.semaphore_wait(second_barrier, 2)


def kernel(x, Wq, Wk, Wv, Wo):
    x2d = x.reshape(CH, D)
    out, _, _ = pl.pallas_call(
        _body,
        out_shape=[
            jax.ShapeDtypeStruct((CH, D), F32),
            jax.ShapeDtypeStruct((SQ, D), BF16),
            jax.ShapeDtypeStruct((N_DEV - 1, CH, D), BF16),
        ],
        in_specs=[pl.BlockSpec(memory_space=pl.ANY)] * 7,
        out_specs=[pl.BlockSpec(memory_space=pltpu.VMEM)]
        + [pl.BlockSpec(memory_space=pl.ANY)] * 2,
        scratch_shapes=[
            pltpu.VMEM((SQ, D), BF16),
            pltpu.VMEM((CH, D), F32),
            pltpu.VMEM((CH, D), BF16),
            pltpu.VMEM((CH, D), BF16),
            pltpu.VMEM((SQ, DH), BF16),
            pltpu.VMEM((SQ, DH), BF16),
            pltpu.VMEM((SQ, DH), BF16),
            pltpu.VMEM((SQ, DH), BF16),
            pltpu.VMEM((SQ, DH), BF16),
            pltpu.VMEM((SQ, DH), BF16),
            pltpu.SemaphoreType.DMA((4,)),
            pltpu.SemaphoreType.DMA((N_DEV - 1,)),
            pltpu.SemaphoreType.DMA((N_DEV - 1,)),
            pltpu.SemaphoreType.DMA((N_DEV - 1,)),
            pltpu.SemaphoreType.DMA((N_DEV - 1,)),
            pltpu.SemaphoreType.DMA((N_DEV - 1,)),
            pltpu.SemaphoreType.DMA((N_DEV - 1,)),
        ],
        compiler_params=pltpu.CompilerParams(collective_id=0),
    )(x2d, Wq, Wk, Wv, Wo, jnp.asarray(_COS), jnp.asarray(_SIN))
    return out.reshape(1, CH, D)


# device time: 352148 ns/iter; 2.5198x vs baseline; 1.0899x over previous
import functools

import jax
import jax.numpy as jnp
import numpy as np
from jax import lax
from jax.experimental import pallas as pl
from jax.experimental.pallas import tpu as pltpu

N_DEV = 4
SQ = 4096
CH = 1024
D = 1024
H = 8
DH = 128
QB = 512
SCALE = 0.08838834764831843

F32 = jnp.float32
BF16 = jnp.bfloat16


def _rope_tables():
    inv = 1.0 / (10000.0 ** (np.arange(0, DH, 2) / DH))
    pos = np.arange(SQ)[:, None] * inv[None, :]
    cos = np.repeat(np.cos(pos), 2, axis=-1).astype(np.float32)
    sin = np.repeat(np.sin(pos), 2, axis=-1).astype(np.float32)
    return cos, sin


_COS, _SIN = _rope_tables()


def _body(x_ref, wq_ref, wk_ref, wv_ref, wo_ref, cos_ref, sin_ref,
          out_ref, ctx_hbm, rcv,
          xf, fb1, hb1, hb2, cosb, sinb, qs, ks, vs, ctxh,
          ldma, ag_ss, ag_rs, al_ss, al_rs, rs_ss, rs_rs, rl_ss, rl_rs):
    my = lax.axis_index("i")
    right = lax.rem(my + 1, N_DEV)
    left = lax.rem(my + N_DEV - 1, N_DEV)

    barrier = pltpu.get_barrier_semaphore()
    for nbr in (left, right):
        pl.semaphore_signal(barrier, inc=1, device_id=(nbr,),
                            device_id_type=pl.DeviceIdType.MESH)
    pl.semaphore_wait(barrier, 2)

    cp = pltpu.make_async_copy(x_ref, fb1, ldma.at[0])
    cp.start()
    cp.wait()
    xf[pl.ds(my * CH, CH), :] = fb1[...].astype(BF16)

    HC = D // 2

    def _ag_hop(hop):
        o_r = lax.rem(my - hop + N_DEV, N_DEV)
        o_l = lax.rem(my + hop, N_DEV)
        r = pltpu.make_async_remote_copy(
            src_ref=xf.at[pl.ds(o_r * CH, CH), pl.ds(0, HC)],
            dst_ref=xf.at[pl.ds(o_r * CH, CH), pl.ds(0, HC)],
            send_sem=ag_ss.at[hop],
            recv_sem=ag_rs.at[hop],
            device_id=(right,),
            device_id_type=pl.DeviceIdType.MESH,
        )
        l = pltpu.make_async_remote_copy(
            src_ref=xf.at[pl.ds(o_l * CH, CH), pl.ds(HC, HC)],
            dst_ref=xf.at[pl.ds(o_l * CH, CH), pl.ds(HC, HC)],
            send_sem=al_ss.at[hop],
            recv_sem=al_rs.at[hop],
            device_id=(left,),
            device_id_type=pl.DeviceIdType.MESH,
        )
        return r, l

    rdma_r, rdma_l = _ag_hop(0)
    rdma_r.start()
    rdma_l.start()
    for c in range(N_DEV):
        csl = pl.ds(c * CH, CH)
        cc = pltpu.make_async_copy(cos_ref.at[csl], fb1.at[:, 0:DH],
                                   ldma.at[1])
        cs = pltpu.make_async_copy(sin_ref.at[csl], fb1.at[:, DH:2 * DH],
                                   ldma.at[2])
        cc.start()
        cs.start()
        cc.wait()
        cs.wait()
        cosb[csl, :] = fb1[:, 0:DH].astype(BF16)
        sinb[csl, :] = fb1[:, DH:2 * DH].astype(BF16)
    rdma_r.wait()
    rdma_l.wait()
    for hop in (1, 2):
        rdma_r, rdma_l = _ag_hop(hop)
        rdma_r.start()
        rdma_l.start()
        rdma_r.wait()
        rdma_l.wait()

    lane = lax.broadcasted_iota(jnp.int32, (CH, DH), 1)
    even = (lane % 2) == 0

    def rope(t, c, s):
        t_rot = jnp.where(even, -pltpu.roll(t, DH - 1, 1),
                          pltpu.roll(t, 1, 1))
        return t * c + t_rot * s

    def head_body(h, carry):
        hsl = pl.ds(h * DH, DH)
        cws = []
        for i, w_ref in enumerate((wq_ref, wk_ref, wv_ref)):
            cw = pltpu.make_async_copy(
                w_ref.at[:, hsl], fb1.at[:, pl.ds(i * DH, DH)], ldma.at[i])
            cw.start()
            cws.append(cw)
        for cw in cws:
            cw.wait()
        wq16 = fb1[:, 0:DH].astype(BF16)
        wk16 = fb1[:, DH:2 * DH].astype(BF16)
        wv16 = fb1[:, 2 * DH:3 * DH].astype(BF16)

        def chunk_body(c, carry):
            csl = pl.ds(c * CH, CH)
            xc = xf[csl, :]
            qc = jnp.dot(xc, wq16, preferred_element_type=F32)
            kc = jnp.dot(xc, wk16, preferred_element_type=F32)
            vc = jnp.dot(xc, wv16, preferred_element_type=F32)
            qs[csl, :] = (rope(qc, cosb[csl, :], sinb[csl, :])
                          * SCALE).astype(BF16)
            ks[csl, :] = rope(kc, cosb[csl, :], sinb[csl, :]).astype(BF16)
            vs[csl, :] = vc.astype(BF16)
            return carry

        lax.fori_loop(0, N_DEV, chunk_body, 0)

        def qb_body(qb, carry):
            qsl = pl.ds(qb * QB, QB)
            s = lax.dot_general(qs[qsl, :], ks[...], (((1,), (1,)), ((), ())),
                                preferred_element_type=F32)
            e = jnp.exp(s)
            denom = jnp.sum(e, axis=1, keepdims=True)
            ctxu = jnp.dot(e.astype(BF16), vs[...],
                           preferred_element_type=F32)
            ctxh[qsl, :] = (ctxu / denom).astype(BF16)
            return carry

        lax.fori_loop(0, SQ // QB, qb_body, 0)
        co = pltpu.make_async_copy(ctxh, ctx_hbm.at[:, hsl], ldma.at[3])
        co.start()
        co.wait()
        return carry

    lax.fori_loop(0, H, head_body, 0)

    cw = pltpu.make_async_copy(wo_ref, fb1, ldma.at[0])
    cw.start()
    cw.wait()
    wo16 = fb1[...].astype(BF16)

    def _half_partial(c, lo):
        ci = pltpu.make_async_copy(ctx_hbm.at[pl.ds(c * CH, CH)], hb1,
                                   ldma.at[1])
        ci.start()
        ci.wait()
        return jnp.dot(hb1[...], wo16[:, lo:lo + HC],
                       preferred_element_type=F32)

    def _step_partials(st):
        d_r = _half_partial(lax.rem(my - 1 - st + N_DEV, N_DEV), 0)
        d_l = _half_partial(lax.rem(my + 1 + st, N_DEV), HC)
        return d_r, d_l

    def _rs_hops(st):
        r = pltpu.make_async_remote_copy(
            src_ref=hb2.at[:, pl.ds(0, HC)],
            dst_ref=rcv.at[st, :, pl.ds(0, HC)],
            send_sem=rs_ss.at[st],
            recv_sem=rs_rs.at[st],
            device_id=(right,),
            device_id_type=pl.DeviceIdType.MESH,
        )
        l = pltpu.make_async_remote_copy(
            src_ref=hb2.at[:, pl.ds(HC, HC)],
            dst_ref=rcv.at[st, :, pl.ds(HC, HC)],
            send_sem=rl_ss.at[st],
            recv_sem=rl_rs.at[st],
            device_id=(left,),
            device_id_type=pl.DeviceIdType.MESH,
        )
        return r, l

    d_r, d_l = _step_partials(0)
    hb2[:, 0:HC] = d_r.astype(BF16)
    hb2[:, HC:D] = d_l.astype(BF16)
    rs_r, rs_l = _rs_hops(0)
    rs_r.start()
    rs_l.start()
    for st in (1, 2):
        d_r, d_l = _step_partials(st)
        rs_r.wait()
        rs_l.wait()
        cr = pltpu.make_async_copy(rcv.at[st - 1], hb2, ldma.at[2])
        cr.start()
        cr.wait()
        hb2[:, 0:HC] = (hb2[:, 0:HC].astype(F32) + d_r).astype(BF16)
        hb2[:, HC:D] = (hb2[:, HC:D].astype(F32) + d_l).astype(BF16)
        rs_r, rs_l = _rs_hops(st)
        rs_r.start()
        rs_l.start()
    d_r = _half_partial(my, 0)
    d_l = _half_partial(my, HC)
    rs_r.wait()
    rs_l.wait()
    cr = pltpu.make_async_copy(rcv.at[N_DEV - 2], hb2, ldma.at[2])
    cr.start()
    cr.wait()
    out_ref[:, 0:HC] = hb2[:, 0:HC].astype(F32) + d_r
    out_ref[:, HC:D] = hb2[:, HC:D].astype(F32) + d_l

    @functools.partial(pl.run_scoped,
                       second_barrier=pltpu.SemaphoreType.REGULAR)
    def _(second_barrier):
        for nbr in (left, right):
            pl.semaphore_signal(second_barrier, inc=1, device_id=(nbr,),
                                device_id_type=pl.DeviceIdType.MESH)
        pl.semaphore_wait(second_barrier, 2)


def kernel(x, Wq, Wk, Wv, Wo):
    x2d = x.reshape(CH, D)
    out, _, _ = pl.pallas_call(
        _body,
        out_shape=[
            jax.ShapeDtypeStruct((CH, D), F32),
            jax.ShapeDtypeStruct((SQ, D), BF16),
            jax.ShapeDtypeStruct((N_DEV - 1, CH, D), BF16),
        ],
        in_specs=[pl.BlockSpec(memory_space=pl.ANY)] * 7,
        out_specs=[pl.BlockSpec(memory_space=pltpu.VMEM)]
        + [pl.BlockSpec(memory_space=pl.ANY)] * 2,
        scratch_shapes=[
            pltpu.VMEM((SQ, D), BF16),
            pltpu.VMEM((CH, D), F32),
            pltpu.VMEM((CH, D), BF16),
            pltpu.VMEM((CH, D), BF16),
            pltpu.VMEM((SQ, DH), BF16),
            pltpu.VMEM((SQ, DH), BF16),
            pltpu.VMEM((SQ, DH), BF16),
            pltpu.VMEM((SQ, DH), BF16),
            pltpu.VMEM((SQ, DH), BF16),
            pltpu.VMEM((SQ, DH), BF16),
            pltpu.SemaphoreType.DMA((4,)),
            pltpu.SemaphoreType.DMA((N_DEV - 1,)),
            pltpu.SemaphoreType.DMA((N_DEV - 1,)),
            pltpu.SemaphoreType.DMA((N_DEV - 1,)),
            pltpu.SemaphoreType.DMA((N_DEV - 1,)),
            pltpu.SemaphoreType.DMA((N_DEV - 1,)),
            pltpu.SemaphoreType.DMA((N_DEV - 1,)),
            pltpu.SemaphoreType.DMA((N_DEV - 1,)),
            pltpu.SemaphoreType.DMA((N_DEV - 1,)),
        ],
        compiler_params=pltpu.CompilerParams(collective_id=0),
    )(x2d, Wq, Wk, Wv, Wo, jnp.asarray(_COS), jnp.asarray(_SIN))
    return out.reshape(1, CH, D)
